# baseline (speedup 1.0000x reference)
import math

import jax
import jax.numpy as jnp
from jax import lax
from jax.experimental import pallas as pl
from jax.experimental.pallas import tpu as pltpu
from jax.experimental.pallas import tpu_sc as plsc

B, C, H, W = 4, 3, 512, 512
NC, NS, L = 2, 16, 16
NW = NC * NS
ROWS = B * H
RPW = ROWS // NW
TPB = H // RPW
RBLK = 8
NBLK = RPW // RBLK
NBI = NBLK // 2
PAD = 48
AW = 576
EPS = 1e-6
LN_BASE = math.log(1.414)


def _min_body(d_ref, o_ref):
    o_ref[...] = jnp.broadcast_to(jnp.min(d_ref[...]), (8, 128))


def _sc_body(im_hbm, disp_hbm, gmin_hbm, res_hbm, occ_hbm,
             disp_v, im_v, acc0, acc1, acc2, acc3, acc4, res_v, occ_v, gmin_v,
             sem_in0, sem_in1, sem_out0, sem_out1):
    accs = (acc0, acc1, acc2, acc3, acc4)
    cid = lax.axis_index("c")
    sid = lax.axis_index("s")
    wid = sid * NC + cid
    b = wid // TPB
    y0 = (wid % TPB) * RPW
    sem_in = (sem_in0, sem_in1)
    sem_out = (sem_out0, sem_out1)

    pltpu.sync_copy(gmin_hbm.at[0], gmin_v)
    gmin = gmin_v[pl.ds(0, L)]
    ZV = jnp.zeros((L,), jnp.float32)
    lane = lax.iota(jnp.int32, L)
    col_base = (lane & 7) * (W // 8)
    rowsel = lane >> 3
    rowoff = rowsel * AW

    def in_copies(s, y):
        cps = [pltpu.make_async_copy(
            disp_hbm.at[b, pl.ds(y, RBLK)], disp_v.at[s], sem_in[s])]
        for c in range(C):
            cps.append(pltpu.make_async_copy(
                im_hbm.at[b, c, pl.ds(y, RBLK)], im_v.at[s, c], sem_in[s]))
        return cps

    def out_copies(s, y):
        cps = []
        for c in range(C):
            cps.append(pltpu.make_async_copy(
                res_v.at[s, c], res_hbm.at[b, c, pl.ds(y, RBLK)], sem_out[s]))
        cps.append(pltpu.make_async_copy(
            occ_v.at[s], occ_hbm.at[b, pl.ds(y, RBLK)], sem_out[s]))
        return cps

    def zero_body(i, c0):
        for a in accs:
            a[pl.ds(i * L, L)] = ZV
        return c0
    lax.fori_loop(0, RBLK * AW // L, zero_body, 0)

    for cp in in_copies(0, y0):
        cp.start()

    def block_pair(bi, carry):
        for h in range(2):
            blk = 2 * bi + h
            y = y0 + blk * RBLK
            s = h
            for cp in in_copies(s, y):
                cp.wait()
            if h == 0:
                for cp in in_copies(1, y + RBLK):
                    cp.start()
            else:
                @pl.when(bi < NBI - 1)
                def _():
                    for cp in in_copies(0, y + RBLK):
                        cp.start()
            @pl.when(bi > 0)
            def _():
                for cp in out_copies(s, y):
                    cp.wait()

            @plsc.parallel_loop(0, (RBLK // 2) * (W // 8), unroll=4)
            def chunk_body(i):
                rp = i >> 6
                j = i & (W // 8 - 1)
                ri = rp * 2 + rowsel
                cj = col_base + ((lane + j) & (W // 8 - 1))
                d = plsc.load_gather(disp_v.at[s], [ri, cj])
                gx = cj.astype(jnp.float32)
                wm = jnp.exp((d - gmin) * LN_BASE)
                t = gx - d + float(PAD)
                xt = t.astype(jnp.int32)
                w1 = t - xt.astype(jnp.float32)
                w0 = 1.0 - w1
                xi = xt + (rowoff + rp * (2 * AW))
                xj = xi + 1
                for c in range(C):
                    v = plsc.load_gather(im_v.at[s, c], [ri, cj]) * wm
                    plsc.addupdate_scatter(accs[c], [xi], v * w0)
                    plsc.addupdate_scatter(accs[c], [xj], v * w1)
                plsc.addupdate_scatter(acc3, [xi], wm * w0)
                plsc.addupdate_scatter(acc3, [xj], wm * w1)
                plsc.addupdate_scatter(acc4, [xi], w0)
                plsc.addupdate_scatter(acc4, [xj], w1)

            @plsc.parallel_loop(0, RBLK * (W // L), unroll=4)
            def fin_body(i):
                r = i >> 5
                k = i & (W // L - 1)
                koff = r * AW + PAD + k * L
                m = acc3[pl.ds(koff, L)]
                inv = 1.0 / jnp.maximum(m, EPS)
                for c in range(C):
                    res_v[s, c, r, pl.ds(k * L, L)] = (
                        accs[c][pl.ds(koff, L)] * inv)
                    accs[c][pl.ds(koff, L)] = ZV
                o = acc4[pl.ds(koff, L)]
                occ_v[s, r, pl.ds(k * L, L)] = 1.0 - jnp.minimum(o, 1.0)
                acc3[pl.ds(koff, L)] = ZV
                acc4[pl.ds(koff, L)] = ZV

            for cp in out_copies(s, y):
                cp.start()
        return carry
    lax.fori_loop(0, NBI, block_pair, 0)

    for s in range(2):
        y = y0 + (NBLK - 2 + s) * RBLK
        for cp in out_copies(s, y):
            cp.wait()


def kernel(im, disp):
    disp3 = disp.reshape(B, H, W)
    gmin = pl.pallas_call(
        _min_body,
        out_shape=jax.ShapeDtypeStruct((8, 128), jnp.float32),
    )(disp.reshape(ROWS, W))

    mesh = plsc.VectorSubcoreMesh(
        core_axis_name="c", subcore_axis_name="s",
        num_cores=NC, num_subcores=NS)
    run = pl.kernel(
        _sc_body,
        out_type=(
            jax.ShapeDtypeStruct((B, C, H, W), jnp.float32),
            jax.ShapeDtypeStruct((B, H, W), jnp.float32),
        ),
        mesh=mesh,
        compiler_params=pltpu.CompilerParams(needs_layout_passes=False),
        scratch_types=[
            pltpu.VMEM((2, RBLK, W), jnp.float32),
            pltpu.VMEM((2, C, RBLK, W), jnp.float32),
            pltpu.VMEM((RBLK * AW,), jnp.float32),
            pltpu.VMEM((RBLK * AW,), jnp.float32),
            pltpu.VMEM((RBLK * AW,), jnp.float32),
            pltpu.VMEM((RBLK * AW,), jnp.float32),
            pltpu.VMEM((RBLK * AW,), jnp.float32),
            pltpu.VMEM((2, C, RBLK, W), jnp.float32),
            pltpu.VMEM((2, RBLK, W), jnp.float32),
            pltpu.VMEM((128,), jnp.float32),
            pltpu.SemaphoreType.DMA,
            pltpu.SemaphoreType.DMA,
            pltpu.SemaphoreType.DMA,
            pltpu.SemaphoreType.DMA,
        ],
    )
    res, occ = run(im, disp3, gmin)
    return res, occ.reshape(B, 1, H, W)


# speedup vs baseline: 1.0904x; 1.0904x over previous
import math

import jax
import jax.numpy as jnp
from jax import lax
from jax.experimental import pallas as pl
from jax.experimental.pallas import tpu as pltpu
from jax.experimental.pallas import tpu_sc as plsc

B, C, H, W = 4, 3, 512, 512
NC, NS, L = 2, 16, 16
NW = NC * NS
ROWS = B * H
RPW = ROWS // NW
TPB = H // RPW
RBLK = 8
NBLK = RPW // RBLK
NBI = NBLK // 2
PAD = 48
AW = 576
EPS = 1e-6
LN_BASE = math.log(1.414)


def _min_body(d_ref, o_ref):
    o_ref[...] = jnp.broadcast_to(jnp.min(d_ref[...]), (8, 128))


def _sc_body(im_hbm, disp_hbm, gmin_hbm, res_hbm, occ_hbm,
             disp_v, im_v, acc0, acc1, acc2, acc3, acc4, res_v, occ_v, gmin_v,
             sem_in0, sem_in1, sem_out0, sem_out1):
    accs = (acc0, acc1, acc2, acc3, acc4)
    cid = lax.axis_index("c")
    sid = lax.axis_index("s")
    wid = sid * NC + cid
    b = wid // TPB
    y0 = (wid % TPB) * RPW
    sem_in = (sem_in0, sem_in1)
    sem_out = (sem_out0, sem_out1)

    pltpu.sync_copy(gmin_hbm.at[0], gmin_v)
    gmin = gmin_v[pl.ds(0, L)]
    ZV = jnp.zeros((L,), jnp.float32)
    lane_f = lax.iota(jnp.int32, L).astype(jnp.float32)

    def in_copies(s, y):
        cps = [pltpu.make_async_copy(
            disp_hbm.at[b, pl.ds(y, RBLK)], disp_v.at[s], sem_in[s])]
        for c in range(C):
            cps.append(pltpu.make_async_copy(
                im_hbm.at[b, c, pl.ds(y, RBLK)], im_v.at[s, c], sem_in[s]))
        return cps

    def out_copies(s, y):
        cps = []
        for c in range(C):
            cps.append(pltpu.make_async_copy(
                res_v.at[s, c], res_hbm.at[b, c, pl.ds(y, RBLK)], sem_out[s]))
        cps.append(pltpu.make_async_copy(
            occ_v.at[s], occ_hbm.at[b, pl.ds(y, RBLK)], sem_out[s]))
        return cps

    def zero_body(i, c0):
        for a in accs:
            a[pl.ds(i * L, L)] = ZV
        return c0
    lax.fori_loop(0, RBLK * AW // L, zero_body, 0)

    for cp in in_copies(0, y0):
        cp.start()

    def block_pair(bi, carry):
        for h in range(2):
            blk = 2 * bi + h
            y = y0 + blk * RBLK
            s = h
            for cp in in_copies(s, y):
                cp.wait()
            if h == 0:
                for cp in in_copies(1, y + RBLK):
                    cp.start()
            else:
                @pl.when(bi < NBI - 1)
                def _():
                    for cp in in_copies(0, y + RBLK):
                        cp.start()
            @pl.when(bi > 0)
            def _():
                for cp in out_copies(s, y):
                    cp.wait()

            @plsc.parallel_loop(0, RBLK * (W // L), unroll=4)
            def chunk_body(i):
                r = i >> 5
                j = i & (W // L - 1)
                base = r * AW
                d = disp_v[s, r, pl.ds(j * L, L)]
                gx = lane_f + (j * L).astype(jnp.float32)
                wm = jnp.exp((d - gmin) * LN_BASE)
                t = gx - d + float(PAD)
                xt = t.astype(jnp.int32)
                w1 = t - xt.astype(jnp.float32)
                w0 = 1.0 - w1
                xi = xt + base
                xj = xi + 1
                for c in range(C):
                    v = im_v[s, c, r, pl.ds(j * L, L)] * wm
                    plsc.addupdate_scatter(accs[c], [xi], v * w0)
                    plsc.addupdate_scatter(accs[c], [xj], v * w1)
                plsc.addupdate_scatter(acc3, [xi], wm * w0)
                plsc.addupdate_scatter(acc3, [xj], wm * w1)
                plsc.addupdate_scatter(acc4, [xi], w0)
                plsc.addupdate_scatter(acc4, [xj], w1)

            @plsc.parallel_loop(0, RBLK * (W // L), unroll=4)
            def fin_body(i):
                r = i >> 5
                k = i & (W // L - 1)
                koff = r * AW + PAD + k * L
                m = acc3[pl.ds(koff, L)]
                inv = 1.0 / jnp.maximum(m, EPS)
                for c in range(C):
                    res_v[s, c, r, pl.ds(k * L, L)] = (
                        accs[c][pl.ds(koff, L)] * inv)
                    accs[c][pl.ds(koff, L)] = ZV
                o = acc4[pl.ds(koff, L)]
                occ_v[s, r, pl.ds(k * L, L)] = 1.0 - jnp.minimum(o, 1.0)
                acc3[pl.ds(koff, L)] = ZV
                acc4[pl.ds(koff, L)] = ZV

            for cp in out_copies(s, y):
                cp.start()
        return carry
    lax.fori_loop(0, NBI, block_pair, 0)

    for s in range(2):
        y = y0 + (NBLK - 2 + s) * RBLK
        for cp in out_copies(s, y):
            cp.wait()


def kernel(im, disp):
    disp3 = disp.reshape(B, H, W)
    gmin = pl.pallas_call(
        _min_body,
        out_shape=jax.ShapeDtypeStruct((8, 128), jnp.float32),
    )(disp.reshape(ROWS, W))

    mesh = plsc.VectorSubcoreMesh(
        core_axis_name="c", subcore_axis_name="s",
        num_cores=NC, num_subcores=NS)
    run = pl.kernel(
        _sc_body,
        out_type=(
            jax.ShapeDtypeStruct((B, C, H, W), jnp.float32),
            jax.ShapeDtypeStruct((B, H, W), jnp.float32),
        ),
        mesh=mesh,
        compiler_params=pltpu.CompilerParams(needs_layout_passes=False),
        scratch_types=[
            pltpu.VMEM((2, RBLK, W), jnp.float32),
            pltpu.VMEM((2, C, RBLK, W), jnp.float32),
            pltpu.VMEM((RBLK * AW,), jnp.float32),
            pltpu.VMEM((RBLK * AW,), jnp.float32),
            pltpu.VMEM((RBLK * AW,), jnp.float32),
            pltpu.VMEM((RBLK * AW,), jnp.float32),
            pltpu.VMEM((RBLK * AW,), jnp.float32),
            pltpu.VMEM((2, C, RBLK, W), jnp.float32),
            pltpu.VMEM((2, RBLK, W), jnp.float32),
            pltpu.VMEM((128,), jnp.float32),
            pltpu.SemaphoreType.DMA,
            pltpu.SemaphoreType.DMA,
            pltpu.SemaphoreType.DMA,
            pltpu.SemaphoreType.DMA,
        ],
    )
    res, occ = run(im, disp3, gmin)
    return res, occ.reshape(B, 1, H, W)
